# Initial kernel scaffold; baseline (speedup 1.0000x reference)
import jax
import jax.numpy as jnp
from jax.experimental import pallas as pl


def kernel(user_emb, poi_emb, time_emb, weather_emb, season_emb, day_weather_emb, month_weather_emb, user_idxs, poi_idxs, time_idxs, season_idxs, weather_now_idxs, weather_day_idxs, weather_month_idxs, e_W_per_edge):
    raise NotImplementedError("write your pallas kernel here")


# speedup vs baseline: 2.7951x; 2.7951x over previous
import jax
import jax.numpy as jnp
from jax import lax
from jax.experimental import pallas as pl
from jax.experimental.pallas import tpu as pltpu
from jax.experimental.pallas import tpu_sc as plsc

N_U = 10000
N_P = 10000
N_T = 168
N_W = 16
N_S = 4
N_D = 366
N_M = 12
E = 320000
D = 128

NT = 16
EPT = E // NT
B = 32
NCH = EPT // B
ROWS_PT = 624
ROWS_TAIL_AT = NT * ROWS_PT
ROWS_TAIL = N_U - ROWS_TAIL_AT

_f32 = jnp.float32
_i32 = jnp.int32


def _zero_rows(ref, nrows, zero16):
    def row(r, carry):
        for g in range(D // 16):
            ref[r, pl.ds(g * 16, 16)] = zero16
        return carry
    lax.fori_loop(0, nrows, row, 0)


def _zero_flat(ref, nwords, zero16):
    def blk(i, carry):
        ref[pl.ds(i * 16, 16)] = zero16
        return carry
    lax.fori_loop(0, nwords // 16, blk, 0)


def _sc_body(uemb, pemb, temb, ew, uix, pix, tix,
             us, ps, ucnt, pcnt, mup,
             acc, hist):
    c = lax.axis_index("c")
    s = lax.axis_index("s")
    zero16 = jnp.zeros((16,), _f32)
    ones16 = jnp.ones((16,), _f32)

    def scoped(iu, ip, it, bu, bp, bt, bw, bm, bmu, ones_b, zo):
        _zero_rows(bm, B, zero16)
        _zero_flat(zo, ROWS_PT + ROWS_TAIL, zero16)
        for k in range(B // 16):
            ones_b[pl.ds(k * 16, 16)] = ones16

        done = 0
        for k in range((ROWS_PT + B - 1) // B):
            n = min(B, ROWS_PT - done)
            pltpu.sync_copy(bm.at[pl.ds(0, n)],
                            acc.at[pl.ds(s * ROWS_PT + done, n)])
            done += n
        pltpu.sync_copy(zo.at[pl.ds(0, ROWS_PT)],
                        hist.at[pl.ds(s * ROWS_PT, ROWS_PT)])

        @pl.when(s == NT - 1)
        def _():
            pltpu.sync_copy(bm.at[pl.ds(0, ROWS_TAIL)],
                            acc.at[pl.ds(ROWS_TAIL_AT, ROWS_TAIL)])
            pltpu.sync_copy(zo.at[pl.ds(0, ROWS_TAIL)],
                            hist.at[pl.ds(ROWS_TAIL_AT, ROWS_TAIL)])
        plsc.subcore_barrier()

        def chunk(i, carry):
            base = s * EPT + i * B
            pltpu.sync_copy(uix.at[pl.ds(base, B)], iu)
            pltpu.sync_copy(pix.at[pl.ds(base, B)], ip)
            pltpu.sync_copy(tix.at[pl.ds(base, B)], it)
            pltpu.sync_copy(temb.at[it], bt)
            pltpu.sync_copy(ew.at[pl.ds(base, B)], bw)
            pltpu.sync_copy(pemb.at[ip], bp)

            @pl.when(c == 0)
            def _():
                def edge(e, carry2):
                    for g in range(D // 16):
                        sl = pl.ds(g * 16, 16)
                        bm[e, sl] = bp[e, sl] + bt[e, sl] + bw[e, sl]
                    return carry2
                lax.fori_loop(0, B, edge, 0)
                pltpu.sync_copy(ones_b, hist.at[iu], add=True)
                pltpu.sync_copy(bm, acc.at[iu], add=True)

            @pl.when(c == 1)
            def _():
                pltpu.sync_copy(uemb.at[iu], bu)

                def edge(e, carry2):
                    for g in range(D // 16):
                        sl = pl.ds(g * 16, 16)
                        uv = bu[e, sl]
                        bm[e, sl] = uv + bt[e, sl] + bw[e, sl]
                        bmu[e, sl] = uv + bp[e, sl]
                    return carry2
                lax.fori_loop(0, B, edge, 0)
                pltpu.sync_copy(bmu, mup.at[pl.ds(base, B)])
                pltpu.sync_copy(ones_b, hist.at[ip], add=True)
                pltpu.sync_copy(bm, acc.at[ip], add=True)
            return carry

        lax.fori_loop(0, NCH, chunk, 0)
        plsc.subcore_barrier()

        def acc_out(sums_hbm, cnt_hbm):
            done = 0
            for k in range((ROWS_PT + B - 1) // B):
                n = min(B, ROWS_PT - done)
                at = s * ROWS_PT + done
                pltpu.sync_copy(acc.at[pl.ds(at, n)], bm.at[pl.ds(0, n)])
                pltpu.sync_copy(bm.at[pl.ds(0, n)],
                                sums_hbm.at[pl.ds(at, n)])
                done += n
            pltpu.sync_copy(hist.at[pl.ds(s * ROWS_PT, ROWS_PT)],
                            zo.at[pl.ds(0, ROWS_PT)])
            pltpu.sync_copy(zo.at[pl.ds(0, ROWS_PT)],
                            cnt_hbm.at[pl.ds(s * ROWS_PT, ROWS_PT)])

            @pl.when(s == NT - 1)
            def _():
                pltpu.sync_copy(acc.at[pl.ds(ROWS_TAIL_AT, ROWS_TAIL)],
                                bm.at[pl.ds(0, ROWS_TAIL)])
                pltpu.sync_copy(bm.at[pl.ds(0, ROWS_TAIL)],
                                sums_hbm.at[pl.ds(ROWS_TAIL_AT, ROWS_TAIL)])
                pltpu.sync_copy(hist.at[pl.ds(ROWS_TAIL_AT, ROWS_TAIL)],
                                zo.at[pl.ds(0, ROWS_TAIL)])
                pltpu.sync_copy(zo.at[pl.ds(0, ROWS_TAIL)],
                                cnt_hbm.at[pl.ds(ROWS_TAIL_AT, ROWS_TAIL)])

        @pl.when(c == 0)
        def _():
            acc_out(us, ucnt)

        @pl.when(c == 1)
        def _():
            acc_out(ps, pcnt)

    pl.run_scoped(
        scoped,
        pltpu.VMEM((B,), _i32),
        pltpu.VMEM((B,), _i32),
        pltpu.VMEM((B,), _i32),
        pltpu.VMEM((B, D), _f32),
        pltpu.VMEM((B, D), _f32),
        pltpu.VMEM((B, D), _f32),
        pltpu.VMEM((B, D), _f32),
        pltpu.VMEM((B, D), _f32),
        pltpu.VMEM((B, D), _f32),
        pltpu.VMEM((B,), _f32),
        pltpu.VMEM((ROWS_PT + ROWS_TAIL,), _f32),
    )


def _sc_scatter_phase(user_emb, poi_emb, time_emb, e_w, uix, pix, tix):
    mesh = plsc.VectorSubcoreMesh(core_axis_name="c", subcore_axis_name="s")
    out_type = (
        jax.ShapeDtypeStruct((N_U, D), _f32),
        jax.ShapeDtypeStruct((N_P, D), _f32),
        jax.ShapeDtypeStruct((N_U,), _f32),
        jax.ShapeDtypeStruct((N_P,), _f32),
        jax.ShapeDtypeStruct((E, D), _f32),
    )
    scratch = [
        pltpu.VMEM_SHARED((N_U, D), _f32),
        pltpu.VMEM_SHARED((N_U,), _f32),
    ]
    f = pl.kernel(_sc_body, out_type=out_type, mesh=mesh,
                  scratch_types=scratch)
    return f(user_emb, poi_emb, time_emb, e_w, uix, pix, tix)


EC = 2560
NSTEP = E // EC


def _a2_body(mup, ti, si, wi, mi, d0, d1, d2,
             tsum, ssum, wsum, dsum, msum,
             tcnt, scnt, wcnt, dcnt, mcnt):
    step = pl.program_id(0)
    m = mup[...]

    def acc2(out, part):
        @pl.when(step == 0)
        def _():
            out[...] = part

        @pl.when(step != 0)
        def _():
            out[...] = out[...] + part

    def tally(oh, sum_out, cnt_out):
        part = jax.lax.dot_general(
            oh, m, (((1,), (0,)), ((), ())), preferred_element_type=_f32)
        acc2(sum_out, part)
        acc2(cnt_out, jnp.sum(oh, axis=1, keepdims=True))

    def onehot(idx_row, nbins):
        rows = lax.broadcasted_iota(_i32, (nbins, EC), 0)
        return (rows == idx_row).astype(_f32)

    tally(onehot(ti[...], N_T), tsum, tcnt)
    tally(onehot(si[...], N_S), ssum, scnt)
    tally(onehot(wi[...], N_W), wsum, wcnt)
    tally(onehot(mi[...], N_M), msum, mcnt)
    ohd = (onehot(d0[...], N_D) + onehot(d1[...], N_D)
           + onehot(d2[...], N_D))
    tally(ohd, dsum, dcnt)


def _small_tables_phase(mup, tix, six, wix, mix, day0, day1, day2):
    out_shape = (
        jax.ShapeDtypeStruct((N_T, D), _f32),
        jax.ShapeDtypeStruct((N_S, D), _f32),
        jax.ShapeDtypeStruct((N_W, D), _f32),
        jax.ShapeDtypeStruct((N_D, D), _f32),
        jax.ShapeDtypeStruct((N_M, D), _f32),
        jax.ShapeDtypeStruct((N_T, 1), _f32),
        jax.ShapeDtypeStruct((N_S, 1), _f32),
        jax.ShapeDtypeStruct((N_W, 1), _f32),
        jax.ShapeDtypeStruct((N_D, 1), _f32),
        jax.ShapeDtypeStruct((N_M, 1), _f32),
    )
    idx2 = lambda a: a.reshape(1, E)
    row_spec = pl.BlockSpec((1, EC), lambda i: (0, i))

    def fix(shape):
        return pl.BlockSpec(shape, lambda i: (0, 0))

    return pl.pallas_call(
        _a2_body,
        grid=(NSTEP,),
        in_specs=[pl.BlockSpec((EC, D), lambda i: (i, 0))] + [row_spec] * 7,
        out_specs=[fix((N_T, D)), fix((N_S, D)), fix((N_W, D)),
                   fix((N_D, D)), fix((N_M, D)),
                   fix((N_T, 1)), fix((N_S, 1)), fix((N_W, 1)),
                   fix((N_D, 1)), fix((N_M, 1))],
        out_shape=out_shape,
    )(mup, idx2(tix), idx2(six), idx2(wix), idx2(mix),
      idx2(day0), idx2(day1), idx2(day2))


def _b1_body(us, ps, ucnt, pcnt, tsum, ssum, wsum, dsum, msum,
             tcnt, scnt, wcnt, dcnt, mcnt,
             unorm, npoi, ntime, nseason, nweather, nday, nmonth):
    eps = 1e-9
    unorm[...] = us[...] / (ucnt[...] + eps)
    npoi[...] = ps[...] / (pcnt[...] + eps)
    ntime[...] = tsum[...] / (tcnt[...] + eps)
    nseason[...] = ssum[...] / (scnt[...] + eps)
    nweather[...] = wsum[...] / (wcnt[...] + eps)
    nday[...] = dsum[...] / (dcnt[...] + eps)
    nmonth[...] = msum[...] / (mcnt[...] + eps)


def _normalize_phase(us, ps, ucnt, pcnt, small):
    out_shape = (
        jax.ShapeDtypeStruct((N_U, D), _f32),
        jax.ShapeDtypeStruct((N_P, D), _f32),
        jax.ShapeDtypeStruct((N_T, D), _f32),
        jax.ShapeDtypeStruct((N_S, D), _f32),
        jax.ShapeDtypeStruct((N_W, D), _f32),
        jax.ShapeDtypeStruct((N_D, D), _f32),
        jax.ShapeDtypeStruct((N_M, D), _f32),
    )
    return pl.pallas_call(_b1_body, out_shape=out_shape)(
        us, ps, ucnt.reshape(N_U, 1), pcnt.reshape(N_P, 1), *small)


RB = 200


def _b2_body(ublk, ufull, ut, out):
    u = ublk[...]
    uf = ufull[...]
    utt = ut[...]
    sq_r = jnp.sum(u * u, axis=1, keepdims=True)
    sq_c = jnp.sum(utt * utt, axis=0, keepdims=True)
    cross = jax.lax.dot_general(
        u, utt, (((1,), (0,)), ((), ())),
        preferred_element_type=_f32)
    d2 = jnp.maximum(sq_r + sq_c - 2.0 * cross, 0.0)
    dist = jnp.sqrt(d2)
    col = lax.broadcasted_iota(_i32, (RB, N_U), 1)
    sel = jnp.zeros((RB, N_U), _f32)
    big = jnp.float32(3.0e38)
    bigi = jnp.int32(2147483647)
    for _ in range(6):
        mn = jnp.min(dist, axis=1, keepdims=True)
        first = jnp.min(jnp.where(dist == mn, col, bigi), axis=1,
                        keepdims=True)
        hit = col == first
        sel = sel + hit.astype(_f32)
        dist = jnp.where(hit, big, dist)
    nbr = jax.lax.dot_general(
        sel, uf, (((1,), (0,)), ((), ())),
        preferred_element_type=_f32,
        precision=jax.lax.Precision.HIGHEST)
    out[...] = nbr * (1.0 / 6.0)


def _aggregation_phase(unorm):
    ut = unorm.T
    grid = (N_U // RB,)
    return pl.pallas_call(
        _b2_body,
        grid=grid,
        in_specs=[
            pl.BlockSpec((RB, D), lambda i: (i, 0)),
            pl.BlockSpec((N_U, D), lambda i: (0, 0)),
            pl.BlockSpec((D, N_U), lambda i: (0, 0)),
        ],
        out_specs=pl.BlockSpec((RB, D), lambda i: (i, 0)),
        out_shape=jax.ShapeDtypeStruct((N_U, D), _f32),
    )(unorm, unorm, ut)


def kernel(user_emb, poi_emb, time_emb, weather_emb, season_emb,
           day_weather_emb, month_weather_emb,
           user_idxs, poi_idxs, time_idxs, season_idxs, weather_now_idxs,
           weather_day_idxs, weather_month_idxs, e_W_per_edge):
    us, ps, ucnt, pcnt, mup = _sc_scatter_phase(
        user_emb, poi_emb, time_emb, e_W_per_edge,
        user_idxs, poi_idxs, time_idxs)
    small = _small_tables_phase(
        mup, time_idxs, season_idxs, weather_now_idxs, weather_month_idxs,
        weather_day_idxs[:, 0], weather_day_idxs[:, 1],
        weather_day_idxs[:, 2])
    (unorm, new_poi, new_time, new_season, new_weather, new_day,
     new_month) = _normalize_phase(us, ps, ucnt, pcnt, small)
    new_user = _aggregation_phase(unorm)
    return (new_user, new_poi, new_time, new_weather, new_season,
            new_day, new_month)


# speedup vs baseline: 4.8743x; 1.7439x over previous
import jax
import jax.numpy as jnp
from jax import lax
from jax.experimental import pallas as pl
from jax.experimental.pallas import tpu as pltpu
from jax.experimental.pallas import tpu_sc as plsc

N_U = 10000
N_P = 10000
N_T = 168
N_W = 16
N_S = 4
N_D = 366
N_M = 12
E = 320000
D = 128

NT = 16
EPT = E // NT
B = 160
NCH = EPT // B
ROWS_PT = 624
ROWS_TAIL_AT = NT * ROWS_PT
ROWS_TAIL = N_U - ROWS_TAIL_AT

_f32 = jnp.float32
_i32 = jnp.int32


def _zero_rows(ref, nrows, zero16):
    def row(r, carry):
        for g in range(D // 16):
            ref[r, pl.ds(g * 16, 16)] = zero16
        return carry
    lax.fori_loop(0, nrows, row, 0)


def _zero_flat(ref, nwords, zero16):
    def blk(i, carry):
        ref[pl.ds(i * 16, 16)] = zero16
        return carry
    lax.fori_loop(0, nwords // 16, blk, 0)


def _sc_body(uemb, pemb, temb, ew, uix, pix, tix,
             us, ps, ucnt, pcnt, mup,
             acc, hist):
    c = lax.axis_index("c")
    s = lax.axis_index("s")
    zero16 = jnp.zeros((16,), _f32)
    ones16 = jnp.ones((16,), _f32)

    iota16 = jnp.arange(16, dtype=_i32)

    def scoped(iu, ip, it, eix, bp, bt, ones_b, zo):
        _zero_rows(bt, B, zero16)
        _zero_flat(zo, ROWS_PT + ROWS_TAIL, zero16)
        for k in range(B // 16):
            ones_b[pl.ds(k * 16, 16)] = ones16

        done = 0
        for k in range((ROWS_PT + B - 1) // B):
            n = min(B, ROWS_PT - done)
            pltpu.sync_copy(bt.at[pl.ds(0, n)],
                            acc.at[pl.ds(s * ROWS_PT + done, n)])
            done += n
        pltpu.sync_copy(zo.at[pl.ds(0, ROWS_PT)],
                        hist.at[pl.ds(s * ROWS_PT, ROWS_PT)])

        @pl.when(s == NT - 1)
        def _():
            pltpu.sync_copy(bt.at[pl.ds(0, ROWS_TAIL)],
                            acc.at[pl.ds(ROWS_TAIL_AT, ROWS_TAIL)])
            pltpu.sync_copy(zo.at[pl.ds(0, ROWS_TAIL)],
                            hist.at[pl.ds(ROWS_TAIL_AT, ROWS_TAIL)])
        plsc.subcore_barrier()

        def chunk(i, carry):
            base = s * EPT + i * B
            pltpu.sync_copy(uix.at[pl.ds(base, B)], iu)
            pltpu.sync_copy(pix.at[pl.ds(base, B)], ip)
            pltpu.sync_copy(tix.at[pl.ds(base, B)], it)
            for k in range(B // 16):
                eix[pl.ds(k * 16, 16)] = iota16 + (base + k * 16)

            @pl.when(c == 0)
            def _():
                pltpu.sync_copy(temb.at[it], bt)
                pltpu.sync_copy(pemb.at[ip], bt, add=True)
                pltpu.sync_copy(ew.at[eix], bt, add=True)
                pltpu.sync_copy(bt, acc.at[iu], add=True)
                pltpu.sync_copy(ones_b, hist.at[iu], add=True)

            @pl.when(c == 1)
            def _():
                pltpu.sync_copy(temb.at[it], bt)
                pltpu.sync_copy(pemb.at[ip], bp)
                pltpu.sync_copy(uemb.at[iu], bt, add=True)
                pltpu.sync_copy(ew.at[eix], bt, add=True)
                pltpu.sync_copy(uemb.at[iu], bp, add=True)
                pltpu.sync_copy(bt, acc.at[ip], add=True)
                pltpu.sync_copy(ones_b, hist.at[ip], add=True)
                pltpu.sync_copy(bp, mup.at[pl.ds(base, B)])
            return carry

        lax.fori_loop(0, NCH, chunk, 0)
        plsc.subcore_barrier()

        def acc_out(sums_hbm, cnt_hbm):
            done = 0
            for k in range((ROWS_PT + B - 1) // B):
                n = min(B, ROWS_PT - done)
                at = s * ROWS_PT + done
                pltpu.sync_copy(acc.at[pl.ds(at, n)], bt.at[pl.ds(0, n)])
                pltpu.sync_copy(bt.at[pl.ds(0, n)],
                                sums_hbm.at[pl.ds(at, n)])
                done += n
            pltpu.sync_copy(hist.at[pl.ds(s * ROWS_PT, ROWS_PT)],
                            zo.at[pl.ds(0, ROWS_PT)])
            pltpu.sync_copy(zo.at[pl.ds(0, ROWS_PT)],
                            cnt_hbm.at[pl.ds(s * ROWS_PT, ROWS_PT)])

            @pl.when(s == NT - 1)
            def _():
                pltpu.sync_copy(acc.at[pl.ds(ROWS_TAIL_AT, ROWS_TAIL)],
                                bt.at[pl.ds(0, ROWS_TAIL)])
                pltpu.sync_copy(bt.at[pl.ds(0, ROWS_TAIL)],
                                sums_hbm.at[pl.ds(ROWS_TAIL_AT, ROWS_TAIL)])
                pltpu.sync_copy(hist.at[pl.ds(ROWS_TAIL_AT, ROWS_TAIL)],
                                zo.at[pl.ds(0, ROWS_TAIL)])
                pltpu.sync_copy(zo.at[pl.ds(0, ROWS_TAIL)],
                                cnt_hbm.at[pl.ds(ROWS_TAIL_AT, ROWS_TAIL)])

        @pl.when(c == 0)
        def _():
            acc_out(us, ucnt)

        @pl.when(c == 1)
        def _():
            acc_out(ps, pcnt)

    pl.run_scoped(
        scoped,
        pltpu.VMEM((B,), _i32),
        pltpu.VMEM((B,), _i32),
        pltpu.VMEM((B,), _i32),
        pltpu.VMEM((B,), _i32),
        pltpu.VMEM((B, D), _f32),
        pltpu.VMEM((B, D), _f32),
        pltpu.VMEM((B,), _f32),
        pltpu.VMEM((ROWS_PT + ROWS_TAIL,), _f32),
    )


def _sc_scatter_phase(user_emb, poi_emb, time_emb, e_w, uix, pix, tix):
    mesh = plsc.VectorSubcoreMesh(core_axis_name="c", subcore_axis_name="s")
    out_type = (
        jax.ShapeDtypeStruct((N_U, D), _f32),
        jax.ShapeDtypeStruct((N_P, D), _f32),
        jax.ShapeDtypeStruct((N_U,), _f32),
        jax.ShapeDtypeStruct((N_P,), _f32),
        jax.ShapeDtypeStruct((E, D), _f32),
    )
    scratch = [
        pltpu.VMEM_SHARED((N_U, D), _f32),
        pltpu.VMEM_SHARED((N_U,), _f32),
    ]
    f = pl.kernel(_sc_body, out_type=out_type, mesh=mesh,
                  scratch_types=scratch)
    return f(user_emb, poi_emb, time_emb, e_w, uix, pix, tix)


EC = 2560
NSTEP = E // EC


def _a2_body(mup, ti, si, wi, mi, d0, d1, d2,
             tsum, ssum, wsum, dsum, msum,
             tcnt, scnt, wcnt, dcnt, mcnt):
    step = pl.program_id(0)
    m = mup[...]

    def acc2(out, part):
        @pl.when(step == 0)
        def _():
            out[...] = part

        @pl.when(step != 0)
        def _():
            out[...] = out[...] + part

    def tally(oh, sum_out, cnt_out):
        part = jax.lax.dot_general(
            oh, m, (((1,), (0,)), ((), ())), preferred_element_type=_f32)
        acc2(sum_out, part)
        acc2(cnt_out, jnp.sum(oh, axis=1, keepdims=True))

    def onehot(idx_row, nbins):
        rows = lax.broadcasted_iota(_i32, (nbins, EC), 0)
        return (rows == idx_row).astype(_f32)

    tally(onehot(ti[...], N_T), tsum, tcnt)
    tally(onehot(si[...], N_S), ssum, scnt)
    tally(onehot(wi[...], N_W), wsum, wcnt)
    tally(onehot(mi[...], N_M), msum, mcnt)
    ohd = (onehot(d0[...], N_D) + onehot(d1[...], N_D)
           + onehot(d2[...], N_D))
    tally(ohd, dsum, dcnt)


def _small_tables_phase(mup, tix, six, wix, mix, day0, day1, day2):
    out_shape = (
        jax.ShapeDtypeStruct((N_T, D), _f32),
        jax.ShapeDtypeStruct((N_S, D), _f32),
        jax.ShapeDtypeStruct((N_W, D), _f32),
        jax.ShapeDtypeStruct((N_D, D), _f32),
        jax.ShapeDtypeStruct((N_M, D), _f32),
        jax.ShapeDtypeStruct((N_T, 1), _f32),
        jax.ShapeDtypeStruct((N_S, 1), _f32),
        jax.ShapeDtypeStruct((N_W, 1), _f32),
        jax.ShapeDtypeStruct((N_D, 1), _f32),
        jax.ShapeDtypeStruct((N_M, 1), _f32),
    )
    idx2 = lambda a: a.reshape(1, E)
    row_spec = pl.BlockSpec((1, EC), lambda i: (0, i))

    def fix(shape):
        return pl.BlockSpec(shape, lambda i: (0, 0))

    return pl.pallas_call(
        _a2_body,
        grid=(NSTEP,),
        in_specs=[pl.BlockSpec((EC, D), lambda i: (i, 0))] + [row_spec] * 7,
        out_specs=[fix((N_T, D)), fix((N_S, D)), fix((N_W, D)),
                   fix((N_D, D)), fix((N_M, D)),
                   fix((N_T, 1)), fix((N_S, 1)), fix((N_W, 1)),
                   fix((N_D, 1)), fix((N_M, 1))],
        out_shape=out_shape,
    )(mup, idx2(tix), idx2(six), idx2(wix), idx2(mix),
      idx2(day0), idx2(day1), idx2(day2))


def _b1_body(us, ps, ucnt, pcnt, tsum, ssum, wsum, dsum, msum,
             tcnt, scnt, wcnt, dcnt, mcnt,
             unorm, npoi, ntime, nseason, nweather, nday, nmonth):
    eps = 1e-9
    unorm[...] = us[...] / (ucnt[...] + eps)
    npoi[...] = ps[...] / (pcnt[...] + eps)
    ntime[...] = tsum[...] / (tcnt[...] + eps)
    nseason[...] = ssum[...] / (scnt[...] + eps)
    nweather[...] = wsum[...] / (wcnt[...] + eps)
    nday[...] = dsum[...] / (dcnt[...] + eps)
    nmonth[...] = msum[...] / (mcnt[...] + eps)


def _normalize_phase(us, ps, ucnt, pcnt, small):
    out_shape = (
        jax.ShapeDtypeStruct((N_U, D), _f32),
        jax.ShapeDtypeStruct((N_P, D), _f32),
        jax.ShapeDtypeStruct((N_T, D), _f32),
        jax.ShapeDtypeStruct((N_S, D), _f32),
        jax.ShapeDtypeStruct((N_W, D), _f32),
        jax.ShapeDtypeStruct((N_D, D), _f32),
        jax.ShapeDtypeStruct((N_M, D), _f32),
    )
    return pl.pallas_call(_b1_body, out_shape=out_shape)(
        us, ps, ucnt.reshape(N_U, 1), pcnt.reshape(N_P, 1), *small)


RB = 200


def _b2_body(ublk, ufull, ut, out):
    u = ublk[...]
    uf = ufull[...]
    utt = ut[...]
    sq_r = jnp.sum(u * u, axis=1, keepdims=True)
    sq_c = jnp.sum(utt * utt, axis=0, keepdims=True)
    cross = jax.lax.dot_general(
        u, utt, (((1,), (0,)), ((), ())),
        preferred_element_type=_f32)
    d2 = jnp.maximum(sq_r + sq_c - 2.0 * cross, 0.0)
    dist = jnp.sqrt(d2)
    col = lax.broadcasted_iota(_i32, (RB, N_U), 1)
    sel = jnp.zeros((RB, N_U), _f32)
    big = jnp.float32(3.0e38)
    bigi = jnp.int32(2147483647)
    for _ in range(6):
        mn = jnp.min(dist, axis=1, keepdims=True)
        first = jnp.min(jnp.where(dist == mn, col, bigi), axis=1,
                        keepdims=True)
        hit = col == first
        sel = sel + hit.astype(_f32)
        dist = jnp.where(hit, big, dist)
    sel_b = sel.astype(jnp.bfloat16)
    uf_hi = uf.astype(jnp.bfloat16)
    uf_lo = (uf - uf_hi.astype(_f32)).astype(jnp.bfloat16)
    dims = (((1,), (0,)), ((), ()))
    nbr = (jax.lax.dot_general(sel_b, uf_hi, dims,
                               preferred_element_type=_f32)
           + jax.lax.dot_general(sel_b, uf_lo, dims,
                                 preferred_element_type=_f32))
    out[...] = nbr * (1.0 / 6.0)


def _aggregation_phase(unorm):
    ut = unorm.T
    grid = (N_U // RB,)
    return pl.pallas_call(
        _b2_body,
        grid=grid,
        in_specs=[
            pl.BlockSpec((RB, D), lambda i: (i, 0)),
            pl.BlockSpec((N_U, D), lambda i: (0, 0)),
            pl.BlockSpec((D, N_U), lambda i: (0, 0)),
        ],
        out_specs=pl.BlockSpec((RB, D), lambda i: (i, 0)),
        out_shape=jax.ShapeDtypeStruct((N_U, D), _f32),
    )(unorm, unorm, ut)


def kernel(user_emb, poi_emb, time_emb, weather_emb, season_emb,
           day_weather_emb, month_weather_emb,
           user_idxs, poi_idxs, time_idxs, season_idxs, weather_now_idxs,
           weather_day_idxs, weather_month_idxs, e_W_per_edge):
    us, ps, ucnt, pcnt, mup = _sc_scatter_phase(
        user_emb, poi_emb, time_emb, e_W_per_edge,
        user_idxs, poi_idxs, time_idxs)
    small = _small_tables_phase(
        mup, time_idxs, season_idxs, weather_now_idxs, weather_month_idxs,
        weather_day_idxs[:, 0], weather_day_idxs[:, 1],
        weather_day_idxs[:, 2])
    (unorm, new_poi, new_time, new_season, new_weather, new_day,
     new_month) = _normalize_phase(us, ps, ucnt, pcnt, small)
    new_user = _aggregation_phase(unorm)
    return (new_user, new_poi, new_time, new_weather, new_season,
            new_day, new_month)


# speedup vs baseline: 5.2418x; 1.0754x over previous
import jax
import jax.numpy as jnp
from jax import lax
from jax.experimental import pallas as pl
from jax.experimental.pallas import tpu as pltpu
from jax.experimental.pallas import tpu_sc as plsc

N_U = 10000
N_P = 10000
N_T = 168
N_W = 16
N_S = 4
N_D = 366
N_M = 12
E = 320000
D = 128

NT = 16
EPT = E // NT
B = 160
NCH = EPT // B
MUP_SPLIT = 62
ROWS_PT = 624
ROWS_TAIL_AT = NT * ROWS_PT
ROWS_TAIL = N_U - ROWS_TAIL_AT

_f32 = jnp.float32
_i32 = jnp.int32


def _zero_rows(ref, nrows, zero16):
    def row(r, carry):
        for g in range(D // 16):
            ref[r, pl.ds(g * 16, 16)] = zero16
        return carry
    lax.fori_loop(0, nrows, row, 0)


def _zero_flat(ref, nwords, zero16):
    def blk(i, carry):
        ref[pl.ds(i * 16, 16)] = zero16
        return carry
    lax.fori_loop(0, nwords // 16, blk, 0)


def _sc_body(uemb, pemb, temb, ew, uix, pix, tix,
             us, ps, ucnt, pcnt, mup,
             acc, hist):
    c = lax.axis_index("c")
    s = lax.axis_index("s")
    zero16 = jnp.zeros((16,), _f32)
    ones16 = jnp.ones((16,), _f32)

    iota16 = jnp.arange(16, dtype=_i32)

    def scoped(iu, ip, it, eix, bp, bt, ones_b, zo):
        _zero_rows(bt, B, zero16)
        _zero_flat(zo, ROWS_PT + ROWS_TAIL, zero16)
        for k in range(B // 16):
            ones_b[pl.ds(k * 16, 16)] = ones16

        done = 0
        for k in range((ROWS_PT + B - 1) // B):
            n = min(B, ROWS_PT - done)
            pltpu.sync_copy(bt.at[pl.ds(0, n)],
                            acc.at[pl.ds(s * ROWS_PT + done, n)])
            done += n
        pltpu.sync_copy(zo.at[pl.ds(0, ROWS_PT)],
                        hist.at[pl.ds(s * ROWS_PT, ROWS_PT)])

        @pl.when(s == NT - 1)
        def _():
            pltpu.sync_copy(bt.at[pl.ds(0, ROWS_TAIL)],
                            acc.at[pl.ds(ROWS_TAIL_AT, ROWS_TAIL)])
            pltpu.sync_copy(zo.at[pl.ds(0, ROWS_TAIL)],
                            hist.at[pl.ds(ROWS_TAIL_AT, ROWS_TAIL)])
        plsc.subcore_barrier()

        def chunk(i, carry):
            base = s * EPT + i * B
            pltpu.sync_copy(uix.at[pl.ds(base, B)], iu)
            pltpu.sync_copy(pix.at[pl.ds(base, B)], ip)
            pltpu.sync_copy(tix.at[pl.ds(base, B)], it)
            for k in range(B // 16):
                eix[pl.ds(k * 16, 16)] = iota16 + (base + k * 16)

            @pl.when(c == 0)
            def _():
                pltpu.sync_copy(temb.at[it], bt)
                pltpu.sync_copy(pemb.at[ip], bt, add=True)
                pltpu.sync_copy(ew.at[eix], bt, add=True)
                pltpu.sync_copy(bt, acc.at[iu], add=True)
                pltpu.sync_copy(ones_b, hist.at[iu], add=True)

            @pl.when(c == 1)
            def _():
                pltpu.sync_copy(temb.at[it], bt)
                pltpu.sync_copy(uemb.at[iu], bt, add=True)
                pltpu.sync_copy(ew.at[eix], bt, add=True)
                pltpu.sync_copy(bt, acc.at[ip], add=True)
                pltpu.sync_copy(ones_b, hist.at[ip], add=True)

            do_mup = jnp.logical_or(
                jnp.logical_and(c == 0, i < MUP_SPLIT),
                jnp.logical_and(c == 1, i >= MUP_SPLIT))

            @pl.when(do_mup)
            def _():
                pltpu.sync_copy(pemb.at[ip], bp)
                pltpu.sync_copy(uemb.at[iu], bp, add=True)
                pltpu.sync_copy(bp, mup.at[pl.ds(base, B)])
            return carry

        lax.fori_loop(0, NCH, chunk, 0)
        plsc.subcore_barrier()

        def acc_out(sums_hbm, cnt_hbm):
            done = 0
            for k in range((ROWS_PT + B - 1) // B):
                n = min(B, ROWS_PT - done)
                at = s * ROWS_PT + done
                pltpu.sync_copy(acc.at[pl.ds(at, n)], bt.at[pl.ds(0, n)])
                pltpu.sync_copy(bt.at[pl.ds(0, n)],
                                sums_hbm.at[pl.ds(at, n)])
                done += n
            pltpu.sync_copy(hist.at[pl.ds(s * ROWS_PT, ROWS_PT)],
                            zo.at[pl.ds(0, ROWS_PT)])
            pltpu.sync_copy(zo.at[pl.ds(0, ROWS_PT)],
                            cnt_hbm.at[pl.ds(s * ROWS_PT, ROWS_PT)])

            @pl.when(s == NT - 1)
            def _():
                pltpu.sync_copy(acc.at[pl.ds(ROWS_TAIL_AT, ROWS_TAIL)],
                                bt.at[pl.ds(0, ROWS_TAIL)])
                pltpu.sync_copy(bt.at[pl.ds(0, ROWS_TAIL)],
                                sums_hbm.at[pl.ds(ROWS_TAIL_AT, ROWS_TAIL)])
                pltpu.sync_copy(hist.at[pl.ds(ROWS_TAIL_AT, ROWS_TAIL)],
                                zo.at[pl.ds(0, ROWS_TAIL)])
                pltpu.sync_copy(zo.at[pl.ds(0, ROWS_TAIL)],
                                cnt_hbm.at[pl.ds(ROWS_TAIL_AT, ROWS_TAIL)])

        @pl.when(c == 0)
        def _():
            acc_out(us, ucnt)

        @pl.when(c == 1)
        def _():
            acc_out(ps, pcnt)

    pl.run_scoped(
        scoped,
        pltpu.VMEM((B,), _i32),
        pltpu.VMEM((B,), _i32),
        pltpu.VMEM((B,), _i32),
        pltpu.VMEM((B,), _i32),
        pltpu.VMEM((B, D), _f32),
        pltpu.VMEM((B, D), _f32),
        pltpu.VMEM((B,), _f32),
        pltpu.VMEM((ROWS_PT + ROWS_TAIL,), _f32),
    )


def _sc_scatter_phase(user_emb, poi_emb, time_emb, e_w, uix, pix, tix):
    mesh = plsc.VectorSubcoreMesh(core_axis_name="c", subcore_axis_name="s")
    out_type = (
        jax.ShapeDtypeStruct((N_U, D), _f32),
        jax.ShapeDtypeStruct((N_P, D), _f32),
        jax.ShapeDtypeStruct((N_U,), _f32),
        jax.ShapeDtypeStruct((N_P,), _f32),
        jax.ShapeDtypeStruct((E, D), _f32),
    )
    scratch = [
        pltpu.VMEM_SHARED((N_U, D), _f32),
        pltpu.VMEM_SHARED((N_U,), _f32),
    ]
    f = pl.kernel(_sc_body, out_type=out_type, mesh=mesh,
                  scratch_types=scratch)
    return f(user_emb, poi_emb, time_emb, e_w, uix, pix, tix)


EC = 2560
NSTEP = E // EC


def _a2_body(mup, ti, si, wi, mi, d0, d1, d2,
             tsum, ssum, wsum, dsum, msum,
             tcnt, scnt, wcnt, dcnt, mcnt):
    step = pl.program_id(0)
    m = mup[...]

    def acc2(out, part):
        @pl.when(step == 0)
        def _():
            out[...] = part

        @pl.when(step != 0)
        def _():
            out[...] = out[...] + part

    def tally(oh, sum_out, cnt_out):
        part = jax.lax.dot_general(
            oh, m, (((1,), (0,)), ((), ())), preferred_element_type=_f32)
        acc2(sum_out, part)
        acc2(cnt_out, jnp.sum(oh, axis=1, keepdims=True))

    def onehot(idx_row, nbins):
        rows = lax.broadcasted_iota(_i32, (nbins, EC), 0)
        return (rows == idx_row).astype(_f32)

    tally(onehot(ti[...], N_T), tsum, tcnt)
    tally(onehot(si[...], N_S), ssum, scnt)
    tally(onehot(wi[...], N_W), wsum, wcnt)
    tally(onehot(mi[...], N_M), msum, mcnt)
    ohd = (onehot(d0[...], N_D) + onehot(d1[...], N_D)
           + onehot(d2[...], N_D))
    tally(ohd, dsum, dcnt)


def _small_tables_phase(mup, tix, six, wix, mix, day0, day1, day2):
    out_shape = (
        jax.ShapeDtypeStruct((N_T, D), _f32),
        jax.ShapeDtypeStruct((N_S, D), _f32),
        jax.ShapeDtypeStruct((N_W, D), _f32),
        jax.ShapeDtypeStruct((N_D, D), _f32),
        jax.ShapeDtypeStruct((N_M, D), _f32),
        jax.ShapeDtypeStruct((N_T, 1), _f32),
        jax.ShapeDtypeStruct((N_S, 1), _f32),
        jax.ShapeDtypeStruct((N_W, 1), _f32),
        jax.ShapeDtypeStruct((N_D, 1), _f32),
        jax.ShapeDtypeStruct((N_M, 1), _f32),
    )
    idx2 = lambda a: a.reshape(1, E)
    row_spec = pl.BlockSpec((1, EC), lambda i: (0, i))

    def fix(shape):
        return pl.BlockSpec(shape, lambda i: (0, 0))

    return pl.pallas_call(
        _a2_body,
        grid=(NSTEP,),
        in_specs=[pl.BlockSpec((EC, D), lambda i: (i, 0))] + [row_spec] * 7,
        out_specs=[fix((N_T, D)), fix((N_S, D)), fix((N_W, D)),
                   fix((N_D, D)), fix((N_M, D)),
                   fix((N_T, 1)), fix((N_S, 1)), fix((N_W, 1)),
                   fix((N_D, 1)), fix((N_M, 1))],
        out_shape=out_shape,
    )(mup, idx2(tix), idx2(six), idx2(wix), idx2(mix),
      idx2(day0), idx2(day1), idx2(day2))


def _b1_body(us, ps, ucnt, pcnt, tsum, ssum, wsum, dsum, msum,
             tcnt, scnt, wcnt, dcnt, mcnt,
             unorm, npoi, ntime, nseason, nweather, nday, nmonth):
    eps = 1e-9
    unorm[...] = us[...] / (ucnt[...] + eps)
    npoi[...] = ps[...] / (pcnt[...] + eps)
    ntime[...] = tsum[...] / (tcnt[...] + eps)
    nseason[...] = ssum[...] / (scnt[...] + eps)
    nweather[...] = wsum[...] / (wcnt[...] + eps)
    nday[...] = dsum[...] / (dcnt[...] + eps)
    nmonth[...] = msum[...] / (mcnt[...] + eps)


def _normalize_phase(us, ps, ucnt, pcnt, small):
    out_shape = (
        jax.ShapeDtypeStruct((N_U, D), _f32),
        jax.ShapeDtypeStruct((N_P, D), _f32),
        jax.ShapeDtypeStruct((N_T, D), _f32),
        jax.ShapeDtypeStruct((N_S, D), _f32),
        jax.ShapeDtypeStruct((N_W, D), _f32),
        jax.ShapeDtypeStruct((N_D, D), _f32),
        jax.ShapeDtypeStruct((N_M, D), _f32),
    )
    return pl.pallas_call(_b1_body, out_shape=out_shape)(
        us, ps, ucnt.reshape(N_U, 1), pcnt.reshape(N_P, 1), *small)


RB = 200


def _b2_body(ublk, ufull, ut, out):
    u = ublk[...]
    uf = ufull[...]
    utt = ut[...]
    sq_r = jnp.sum(u * u, axis=1, keepdims=True)
    sq_c = jnp.sum(utt * utt, axis=0, keepdims=True)
    cross = jax.lax.dot_general(
        u, utt, (((1,), (0,)), ((), ())),
        preferred_element_type=_f32)
    d2 = jnp.maximum(sq_r + sq_c - 2.0 * cross, 0.0)
    dist = jnp.sqrt(d2)
    col = lax.broadcasted_iota(_i32, (RB, N_U), 1)
    sel = jnp.zeros((RB, N_U), _f32)
    big = jnp.float32(3.0e38)
    bigi = jnp.int32(2147483647)
    for _ in range(6):
        mn = jnp.min(dist, axis=1, keepdims=True)
        first = jnp.min(jnp.where(dist == mn, col, bigi), axis=1,
                        keepdims=True)
        hit = col == first
        sel = sel + hit.astype(_f32)
        dist = jnp.where(hit, big, dist)
    sel_b = sel.astype(jnp.bfloat16)
    uf_hi = uf.astype(jnp.bfloat16)
    uf_lo = (uf - uf_hi.astype(_f32)).astype(jnp.bfloat16)
    dims = (((1,), (0,)), ((), ()))
    nbr = (jax.lax.dot_general(sel_b, uf_hi, dims,
                               preferred_element_type=_f32)
           + jax.lax.dot_general(sel_b, uf_lo, dims,
                                 preferred_element_type=_f32))
    out[...] = nbr * (1.0 / 6.0)


def _aggregation_phase(unorm):
    ut = unorm.T
    grid = (N_U // RB,)
    return pl.pallas_call(
        _b2_body,
        grid=grid,
        in_specs=[
            pl.BlockSpec((RB, D), lambda i: (i, 0)),
            pl.BlockSpec((N_U, D), lambda i: (0, 0)),
            pl.BlockSpec((D, N_U), lambda i: (0, 0)),
        ],
        out_specs=pl.BlockSpec((RB, D), lambda i: (i, 0)),
        out_shape=jax.ShapeDtypeStruct((N_U, D), _f32),
    )(unorm, unorm, ut)


def kernel(user_emb, poi_emb, time_emb, weather_emb, season_emb,
           day_weather_emb, month_weather_emb,
           user_idxs, poi_idxs, time_idxs, season_idxs, weather_now_idxs,
           weather_day_idxs, weather_month_idxs, e_W_per_edge):
    us, ps, ucnt, pcnt, mup = _sc_scatter_phase(
        user_emb, poi_emb, time_emb, e_W_per_edge,
        user_idxs, poi_idxs, time_idxs)
    small = _small_tables_phase(
        mup, time_idxs, season_idxs, weather_now_idxs, weather_month_idxs,
        weather_day_idxs[:, 0], weather_day_idxs[:, 1],
        weather_day_idxs[:, 2])
    (unorm, new_poi, new_time, new_season, new_weather, new_day,
     new_month) = _normalize_phase(us, ps, ucnt, pcnt, small)
    new_user = _aggregation_phase(unorm)
    return (new_user, new_poi, new_time, new_weather, new_season,
            new_day, new_month)
